# Initial kernel scaffold; baseline (speedup 1.0000x reference)
import jax
import jax.numpy as jnp
from jax.experimental import pallas as pl


def kernel(x1, edge_index1, x2, edge_index2, W0, b0, W1, b1):
    raise NotImplementedError("write your pallas kernel here")


# speedup vs baseline: 11.3843x; 11.3843x over previous
import functools

import jax
import jax.numpy as jnp
from jax import lax
from jax.experimental import pallas as pl
from jax.experimental.pallas import tpu as pltpu
from jax.experimental.pallas import tpu_sc as plsc

N = 10000
E = 320000
D = 128
NS = 16
L = 16
K = 128
IB = 16
NBLK = 10
NCHUNKS = IB * NBLK
EPAD = NS * NCHUNKS * K
NPAD = 10240
DUMMY = 10100
STRIPE = NPAD // NS
ZR = 64
BR = 1000

_mesh = plsc.VectorSubcoreMesh(core_axis_name="c", subcore_axis_name="s")


def _fill_zeros(ref, rows, cols):
    def body(i, _):
        r = i // (cols // L)
        q = i % (cols // L)
        ref[r, pl.ds(q * L, L)] = jnp.zeros((L,), jnp.float32)
        return 0

    lax.fori_loop(0, rows * (cols // L), body, 0)


@functools.partial(
    pl.kernel,
    out_type=[jax.ShapeDtypeStruct((NPAD, L), jnp.float32)] * 2,
    mesh=_mesh,
    scratch_types=[
        pltpu.VMEM((NCHUNKS, K), jnp.int32),
        pltpu.VMEM((K, L), jnp.float32),
        pltpu.VMEM((ZR, L), jnp.float32),
        pltpu.VMEM_SHARED((NPAD, L), jnp.float32),
    ],
)
def _deg_sc(dst1, dst2, deg1, deg2, dstv, ones_v, zv, acc):
    c = lax.axis_index("c")
    s = lax.axis_index("s")
    base = s * STRIPE

    _fill_zeros(zv, ZR, L)

    def fill_ones(i, _):
        ones_v[i, pl.ds(0, L)] = jnp.ones((L,), jnp.float32)
        return 0

    lax.fori_loop(0, K, fill_ones, 0)

    def zcp(i, _):
        pltpu.sync_copy(zv, acc.at[pl.ds(base + i * ZR, ZR)])
        return 0

    lax.fori_loop(0, STRIPE // ZR, zcp, 0)
    plsc.subcore_barrier()

    def run(dsth):
        pltpu.sync_copy(dsth.at[s], dstv)

        def body(j, _):
            pltpu.sync_copy(ones_v, acc.at[dstv.at[j]], add=True)
            return 0

        lax.fori_loop(0, NCHUNKS, body, 0)

    @pl.when(c == 0)
    def _():
        run(dst1)

    @pl.when(c == 1)
    def _():
        run(dst2)

    plsc.subcore_barrier()

    def out_to(outh):
        def ocp(i, _):
            pltpu.sync_copy(acc.at[pl.ds(base + i * K, K)],
                            outh.at[pl.ds(base + i * K, K)])
            return 0

        lax.fori_loop(0, STRIPE // K, ocp, 0)

    @pl.when(c == 0)
    def _():
        out_to(deg1)

    @pl.when(c == 1)
    def _():
        out_to(deg2)


@functools.partial(
    pl.kernel,
    out_type=[jax.ShapeDtypeStruct((NPAD, D), jnp.float32)] * 2,
    mesh=_mesh,
    scratch_types=[
        pltpu.VMEM((IB, K), jnp.int32),
        pltpu.VMEM((IB, K), jnp.int32),
        pltpu.VMEM((K, D), jnp.float32),
        pltpu.VMEM((K, D), jnp.float32),
        pltpu.VMEM((ZR, D), jnp.float32),
        pltpu.VMEM_SHARED((NPAD, D), jnp.float32),
        pltpu.SemaphoreType.DMA,
        pltpu.SemaphoreType.DMA,
    ],
)
def _msg_sc(hs1, src1, dst1, hs2, src2, dst2, out1, out2,
            sidx, didx, rows0, rows1, zv, acc, semA, semB):
    c = lax.axis_index("c")
    s = lax.axis_index("s")
    base = s * STRIPE

    _fill_zeros(zv, ZR, D)

    def zcp(i, _):
        pltpu.sync_copy(zv, acc.at[pl.ds(base + i * ZR, ZR)])
        return 0

    lax.fori_loop(0, STRIPE // ZR, zcp, 0)
    plsc.subcore_barrier()

    def run(hsh, srch, dsth):
        def gather(j, rbuf, sem):
            return pltpu.make_async_copy(hsh.at[sidx.at[j]], rbuf, sem)

        for b in range(NBLK):
            pltpu.sync_copy(srch.at[s, pl.ds(b * IB, IB)], sidx)
            pltpu.sync_copy(dsth.at[s, pl.ds(b * IB, IB)], didx)
            gather(0, rows0, semA).start()

            def pair(p, _):
                j0 = 2 * p
                gather(j0 + 1, rows1, semB).start()
                gather(j0, rows0, semA).wait()
                pltpu.sync_copy(rows0, acc.at[didx.at[j0]], add=True)

                @pl.when(j0 + 2 < IB)
                def _():
                    gather(j0 + 2, rows0, semA).start()

                gather(j0 + 1, rows1, semB).wait()
                pltpu.sync_copy(rows1, acc.at[didx.at[j0 + 1]], add=True)
                return 0

            lax.fori_loop(0, IB // 2, pair, 0)

    @pl.when(c == 0)
    def _():
        run(hs1, src1, dst1)

    @pl.when(c == 1)
    def _():
        run(hs2, src2, dst2)

    plsc.subcore_barrier()

    def out_to(outh):
        def ocp(i, _):
            pltpu.sync_copy(acc.at[pl.ds(base + i * K, K)],
                            outh.at[pl.ds(base + i * K, K)])
            return 0

        lax.fori_loop(0, STRIPE // K, ocp, 0)

    @pl.when(c == 0)
    def _():
        out_to(out1)

    @pl.when(c == 1)
    def _():
        out_to(out2)


def _dinv_of(deg_blk):
    return lax.rsqrt(deg_blk[:, 0:1] + 1.0)


def _mm_scale_body(x_ref, w_ref, deg_ref, o_ref):
    dinv = _dinv_of(deg_ref[...])
    o_ref[...] = dinv * jnp.dot(x_ref[...], w_ref[...],
                                preferred_element_type=jnp.float32)


def _mm_scale(x, w, deg):
    return pl.pallas_call(
        _mm_scale_body,
        grid=(N // BR,),
        in_specs=[
            pl.BlockSpec((BR, D), lambda i: (i, 0)),
            pl.BlockSpec((D, D), lambda i: (0, 0)),
            pl.BlockSpec((BR, L), lambda i: (i, 0)),
        ],
        out_specs=pl.BlockSpec((BR, D), lambda i: (i, 0)),
        out_shape=jax.ShapeDtypeStruct((N, D), jnp.float32),
    )(x, w, deg)


def _layer1_body(hs_ref, s_ref, deg_ref, b_ref, w_ref, o_ref):
    dinv = _dinv_of(deg_ref[...])
    r = jnp.maximum(dinv * (hs_ref[...] + s_ref[...]) + b_ref[...], 0.0)
    o_ref[...] = dinv * jnp.dot(r, w_ref[...], preferred_element_type=jnp.float32)


def _layer1(hs0, s0, deg, b0, w1):
    return pl.pallas_call(
        _layer1_body,
        grid=(N // BR,),
        in_specs=[
            pl.BlockSpec((BR, D), lambda i: (i, 0)),
            pl.BlockSpec((BR, D), lambda i: (i, 0)),
            pl.BlockSpec((BR, L), lambda i: (i, 0)),
            pl.BlockSpec((1, D), lambda i: (0, 0)),
            pl.BlockSpec((D, D), lambda i: (0, 0)),
        ],
        out_specs=pl.BlockSpec((BR, D), lambda i: (i, 0)),
        out_shape=jax.ShapeDtypeStruct((N, D), jnp.float32),
    )(hs0, s0, deg, b0, w1)


def _final_body(hs_ref, s_ref, deg_ref, b_ref, h_ref, st_ref, sc_ref):
    i = pl.program_id(0)
    dinv = _dinv_of(deg_ref[...])
    h = dinv * (hs_ref[...] + s_ref[...]) + b_ref[...]
    h_ref[...] = h

    @pl.when(i == 0)
    def _():
        sc_ref[...] = jnp.zeros_like(sc_ref)

    sc_ref[0:1, :] += jnp.sum(h, axis=0, keepdims=True)
    sc_ref[1:2, :] += jnp.sum(h * h, axis=0, keepdims=True)

    @pl.when(i == pl.num_programs(0) - 1)
    def _():
        st_ref[...] = sc_ref[...]


def _final(hs1, s1, deg, b1):
    return pl.pallas_call(
        _final_body,
        grid=(N // BR,),
        in_specs=[
            pl.BlockSpec((BR, D), lambda i: (i, 0)),
            pl.BlockSpec((BR, D), lambda i: (i, 0)),
            pl.BlockSpec((BR, L), lambda i: (i, 0)),
            pl.BlockSpec((1, D), lambda i: (0, 0)),
        ],
        out_specs=[
            pl.BlockSpec((BR, D), lambda i: (i, 0)),
            pl.BlockSpec((8, D), lambda i: (0, 0)),
        ],
        out_shape=[
            jax.ShapeDtypeStruct((N, D), jnp.float32),
            jax.ShapeDtypeStruct((8, D), jnp.float32),
        ],
        scratch_shapes=[pltpu.VMEM((8, D), jnp.float32)],
    )(hs1, s1, deg, b1)


def _norm_body(h_ref, st_ref, z_ref):
    s = st_ref[0:1, :]
    s2 = st_ref[1:2, :]
    mean = s * (1.0 / N)
    var = (s2 - mean * mean * N) * (1.0 / (N - 1))
    z_ref[...] = (h_ref[...] - mean) * lax.rsqrt(var)


def _normalize(h, st):
    return pl.pallas_call(
        _norm_body,
        grid=(N // BR,),
        in_specs=[
            pl.BlockSpec((BR, D), lambda i: (i, 0)),
            pl.BlockSpec((8, D), lambda i: (0, 0)),
        ],
        out_specs=pl.BlockSpec((BR, D), lambda i: (i, 0)),
        out_shape=jax.ShapeDtypeStruct((N, D), jnp.float32),
    )(h, st)


def _prep_edges(edge_index):
    src = edge_index[0]
    dst = edge_index[1]
    pad = EPAD - E
    src = jnp.concatenate([src, jnp.zeros((pad,), jnp.int32)])
    dst = jnp.concatenate([dst, jnp.full((pad,), DUMMY, jnp.int32)])
    return src.reshape(NS, NCHUNKS, K), dst.reshape(NS, NCHUNKS, K)


def kernel(x1, edge_index1, x2, edge_index2, W0, b0, W1, b1):
    s1, d1 = _prep_edges(edge_index1)
    s2, d2 = _prep_edges(edge_index2)
    b0r = b0.reshape(1, D)
    b1r = b1.reshape(1, D)

    deg1, deg2 = _deg_sc(d1, d2)

    hs0_1 = _mm_scale(x1, W0, deg1)
    hs0_2 = _mm_scale(x2, W0, deg2)
    acc0_1, acc0_2 = _msg_sc(hs0_1, s1, d1, hs0_2, s2, d2)

    hs1_1 = _layer1(hs0_1, acc0_1, deg1, b0r, W1)
    hs1_2 = _layer1(hs0_2, acc0_2, deg2, b0r, W1)
    acc1_1, acc1_2 = _msg_sc(hs1_1, s1, d1, hs1_2, s2, d2)

    h1, st1 = _final(hs1_1, acc1_1, deg1, b1r)
    h2, st2 = _final(hs1_2, acc1_2, deg2, b1r)

    z1 = _normalize(h1, st1)
    z2 = _normalize(h2, st2)
    return (z1, z2)
